# Initial kernel scaffold; baseline (speedup 1.0000x reference)
import jax
import jax.numpy as jnp
from jax.experimental import pallas as pl


def kernel(x, router_w, u, svh, pretrained_w):
    raise NotImplementedError("write your pallas kernel here")


# speedup vs baseline: 4.5936x; 4.5936x over previous
import functools

import jax
import jax.numpy as jnp
from jax.experimental import pallas as pl
from jax.experimental.pallas import tpu as pltpu

_IN = 2048
_OUT = 2048
_NC = 1024
_TOPK = 8
_TOKENS = 8192
_TILE = 256


def _body(x_ref, rw_ref, u_ref, svh_ref, pw_ref, o_ref):
    xb = x_ref[...]

    logits = jax.lax.dot_general(
        xb, rw_ref[...], (((1,), (1,)), ((), ())),
        preferred_element_type=jnp.float32,
        precision=jax.lax.Precision.HIGHEST,
    )
    a = jnp.abs(logits)

    cur = a
    thr = jnp.zeros((a.shape[0], 1), jnp.float32)
    for _ in range(_TOPK):
        thr = jnp.max(cur, axis=1, keepdims=True)
        cur = jnp.where(cur >= thr, -jnp.inf, cur)

    dots = jax.lax.dot_general(
        xb, svh_ref[...], (((1,), (1,)), ((), ())),
        preferred_element_type=jnp.float32,
        precision=jax.lax.Precision.HIGHEST,
    )
    masked = jnp.where(a >= thr, dots, 0.0)

    base = jax.lax.dot_general(
        xb, pw_ref[...], (((1,), (1,)), ((), ())),
        preferred_element_type=jnp.float32,
        precision=jax.lax.Precision.HIGHEST,
    )
    expert = jax.lax.dot_general(
        masked, u_ref[...], (((1,), (1,)), ((), ())),
        preferred_element_type=jnp.float32,
        precision=jax.lax.Precision.HIGHEST,
    )
    o_ref[...] = base + expert


@jax.jit
def kernel(x, router_w, u, svh, pretrained_w):
    grid = (_TOKENS // _TILE,)
    return pl.pallas_call(
        _body,
        grid=grid,
        in_specs=[
            pl.BlockSpec((_TILE, _IN), lambda i: (i, 0)),
            pl.BlockSpec((_NC, _IN), lambda i: (0, 0)),
            pl.BlockSpec((_OUT, _NC), lambda i: (0, 0)),
            pl.BlockSpec((_NC, _IN), lambda i: (0, 0)),
            pl.BlockSpec((_OUT, _IN), lambda i: (0, 0)),
        ],
        out_specs=pl.BlockSpec((_TILE, _OUT), lambda i: (i, 0)),
        out_shape=jax.ShapeDtypeStruct((_TOKENS, _OUT), jnp.float32),
        compiler_params=pltpu.CompilerParams(
            dimension_semantics=("arbitrary",),
            vmem_limit_bytes=100 * 1024 * 1024,
        ),
    )(x, router_w, u, svh, pretrained_w)


# speedup vs baseline: 23.8592x; 5.1940x over previous
import functools

import jax
import jax.numpy as jnp
from jax.experimental import pallas as pl
from jax.experimental.pallas import tpu as pltpu

_IN = 2048
_OUT = 2048
_NC = 1024
_TOPK = 8
_TOKENS = 8192
_TILE = 256


def _body(x_ref, rw_ref, u_ref, svh_ref, pw_ref, o_ref):
    xb = x_ref[...]

    logits = jax.lax.dot_general(
        xb, rw_ref[...], (((1,), (1,)), ((), ())),
        preferred_element_type=jnp.float32,
        precision=jax.lax.Precision.DEFAULT,
    )
    a = jnp.abs(logits)

    cur = a
    thr = jnp.zeros((a.shape[0], 1), jnp.float32)
    for _ in range(_TOPK):
        thr = jnp.max(cur, axis=1, keepdims=True)
        cur = jnp.where(cur >= thr, -jnp.inf, cur)

    dots = jax.lax.dot_general(
        xb, svh_ref[...], (((1,), (1,)), ((), ())),
        preferred_element_type=jnp.float32,
        precision=jax.lax.Precision.DEFAULT,
    )
    masked = jnp.where(a >= thr, dots, 0.0)

    base = jax.lax.dot_general(
        xb, pw_ref[...], (((1,), (1,)), ((), ())),
        preferred_element_type=jnp.float32,
        precision=jax.lax.Precision.DEFAULT,
    )
    expert = jax.lax.dot_general(
        masked, u_ref[...], (((1,), (1,)), ((), ())),
        preferred_element_type=jnp.float32,
        precision=jax.lax.Precision.DEFAULT,
    )
    o_ref[...] = base + expert


@jax.jit
def kernel(x, router_w, u, svh, pretrained_w):
    grid = (_TOKENS // _TILE,)
    return pl.pallas_call(
        _body,
        grid=grid,
        in_specs=[
            pl.BlockSpec((_TILE, _IN), lambda i: (i, 0)),
            pl.BlockSpec((_NC, _IN), lambda i: (0, 0)),
            pl.BlockSpec((_OUT, _NC), lambda i: (0, 0)),
            pl.BlockSpec((_NC, _IN), lambda i: (0, 0)),
            pl.BlockSpec((_OUT, _IN), lambda i: (0, 0)),
        ],
        out_specs=pl.BlockSpec((_TILE, _OUT), lambda i: (i, 0)),
        out_shape=jax.ShapeDtypeStruct((_TOKENS, _OUT), jnp.float32),
        compiler_params=pltpu.CompilerParams(
            dimension_semantics=("arbitrary",),
            vmem_limit_bytes=100 * 1024 * 1024,
        ),
    )(x, router_w, u, svh, pretrained_w)
